# baseline (speedup 1.0000x reference)
import jax
import jax.numpy as jnp
from jax import lax
from jax.experimental import pallas as pl
from jax.experimental.pallas import tpu as pltpu
from jax.experimental.pallas import tpu_sc as plsc

N, E, D = 10000, 320000, 128
EPS = 1e-5
ROWS = 2000
CHUNK = 40
ZC = 200
LANES = 16


def _lin_body(x_ref, w_ref, b_ref, k_ref, qv_ref, s_ref):
    acc = jnp.dot(x_ref[...], w_ref[...],
                  preferred_element_type=jnp.float32) + b_ref[...]
    k_ref[...] = acc[:, 0:D]
    qlo = lax.bitcast_convert_type(acc[:, D:2 * D].astype(jnp.bfloat16),
                                   jnp.uint16).astype(jnp.int32)
    vhi = lax.bitcast_convert_type(acc[:, 2 * D:3 * D].astype(jnp.bfloat16),
                                   jnp.uint16).astype(jnp.int32)
    qv_ref[...] = (vhi << 16) | qlo
    s_ref[...] = acc[:, 3 * D:4 * D]


_LIN_OUT_SPECS = [
    pl.BlockSpec((ROWS, D), lambda i: (i, 0)),
    pl.BlockSpec((ROWS, D), lambda i: (i, 0)),
    pl.BlockSpec((ROWS, D), lambda i: (i, 0)),
]
_LIN_OUT_SHAPE = [
    jax.ShapeDtypeStruct((N, D), jnp.float32),
    jax.ShapeDtypeStruct((N, D), jnp.int32),
    jax.ShapeDtypeStruct((N, D), jnp.float32),
]


def _linear4(x, wcat, bcat):
    return pl.pallas_call(
        _lin_body,
        grid=(N // ROWS,),
        in_specs=[
            pl.BlockSpec((ROWS, D), lambda i: (i, 0)),
            pl.BlockSpec((D, 4 * D), lambda i: (0, 0)),
            pl.BlockSpec((1, 4 * D), lambda i: (0, 0)),
        ],
        out_specs=_LIN_OUT_SPECS,
        out_shape=_LIN_OUT_SHAPE,
    )(x, wcat, bcat)


def _stats_body(a_ref, s_ref, y_ref, sum_ref, sq_ref):
    y = jnp.maximum(a_ref[0] + a_ref[1] + s_ref[...], 0.0)
    y_ref[...] = y

    @pl.when(pl.program_id(0) == 0)
    def _():
        sum_ref[...] = jnp.zeros_like(sum_ref)
        sq_ref[...] = jnp.zeros_like(sq_ref)

    sum_ref[...] += jnp.broadcast_to(jnp.sum(y, axis=0), (8, D))
    sq_ref[...] += jnp.broadcast_to(jnp.sum(y * y, axis=0), (8, D))


def _relu_stats(agg_parts, s):
    return pl.pallas_call(
        _stats_body,
        grid=(N // ROWS,),
        in_specs=[
            pl.BlockSpec((2, ROWS, D), lambda i: (0, i, 0)),
            pl.BlockSpec((ROWS, D), lambda i: (i, 0)),
        ],
        out_specs=[
            pl.BlockSpec((ROWS, D), lambda i: (i, 0)),
            pl.BlockSpec((8, D), lambda i: (0, 0)),
            pl.BlockSpec((8, D), lambda i: (0, 0)),
        ],
        out_shape=[
            jax.ShapeDtypeStruct((N, D), jnp.float32),
            jax.ShapeDtypeStruct((8, D), jnp.float32),
            jax.ShapeDtypeStruct((8, D), jnp.float32),
        ],
    )(agg_parts, s)


def _bn(y_ref, sum_ref, sq_ref, g_ref, be_ref):
    mean = sum_ref[0:1, :] * (1.0 / N)
    var = sq_ref[0:1, :] * (1.0 / N) - mean * mean
    inv = lax.rsqrt(var + EPS)
    return (y_ref[...] - mean) * (inv * g_ref[...]) + be_ref[...]


def _bn_lin_body(y_ref, sum_ref, sq_ref, g_ref, be_ref, w_ref, b_ref,
                 k_ref, qv_ref, s_ref):
    h = _bn(y_ref, sum_ref, sq_ref, g_ref, be_ref)
    acc = jnp.dot(h, w_ref[...], preferred_element_type=jnp.float32) + b_ref[...]
    k_ref[...] = acc[:, 0:D]
    qlo = lax.bitcast_convert_type(acc[:, D:2 * D].astype(jnp.bfloat16),
                                   jnp.uint16).astype(jnp.int32)
    vhi = lax.bitcast_convert_type(acc[:, 2 * D:3 * D].astype(jnp.bfloat16),
                                   jnp.uint16).astype(jnp.int32)
    qv_ref[...] = (vhi << 16) | qlo
    s_ref[...] = acc[:, 3 * D:4 * D]


def _bn_linear4(y, ssum, ssq, g, be, wcat, bcat):
    return pl.pallas_call(
        _bn_lin_body,
        grid=(N // ROWS,),
        in_specs=[
            pl.BlockSpec((ROWS, D), lambda i: (i, 0)),
            pl.BlockSpec((8, D), lambda i: (0, 0)),
            pl.BlockSpec((8, D), lambda i: (0, 0)),
            pl.BlockSpec((1, D), lambda i: (0, 0)),
            pl.BlockSpec((1, D), lambda i: (0, 0)),
            pl.BlockSpec((D, 4 * D), lambda i: (0, 0)),
            pl.BlockSpec((1, 4 * D), lambda i: (0, 0)),
        ],
        out_specs=_LIN_OUT_SPECS,
        out_shape=_LIN_OUT_SHAPE,
    )(y, ssum, ssq, g, be, wcat, bcat)


def _bn_body(y_ref, sum_ref, sq_ref, g_ref, be_ref, out_ref):
    out_ref[...] = _bn(y_ref, sum_ref, sq_ref, g_ref, be_ref)


def _bn_final(y, ssum, ssq, g, be):
    return pl.pallas_call(
        _bn_body,
        grid=(N // ROWS,),
        in_specs=[
            pl.BlockSpec((ROWS, D), lambda i: (i, 0)),
            pl.BlockSpec((8, D), lambda i: (0, 0)),
            pl.BlockSpec((8, D), lambda i: (0, 0)),
            pl.BlockSpec((1, D), lambda i: (0, 0)),
            pl.BlockSpec((1, D), lambda i: (0, 0)),
        ],
        out_specs=pl.BlockSpec((ROWS, D), lambda i: (i, 0)),
        out_shape=jax.ShapeDtypeStruct((N, D), jnp.float32),
    )(y, ssum, ssq, g, be)


def _make_edge_kernel(nc, ns):
    npw = nc * ns
    epw = E // npw
    nchunk = epw // CHUNK
    nsuper = nchunk // 2
    nblk = N // ZC
    bps = (nblk + ns - 1) // ns
    nzblk = N // (2 * CHUNK)
    zps = (nzblk + ns - 1) // ns

    def body(src_hbm, dst_hbm, k_hbm, qv_hbm, out_hbm,
             sidx0, didx0, xdix0, sidx1, didx1, xdix1,
             kb0, qvb0, kb1, qvb1, mb0, mb1, agg,
             gsem0, gsem1, msem0, msem1, isem0, isem1, xsem0, xsem1):
        c = lax.axis_index("c")
        s = lax.axis_index("s")
        wid = s * nc + c
        ebase = wid * epw

        def zrow(r, _):
            for g in range(D // LANES):
                mb0[r, pl.ds(g * LANES, LANES)] = jnp.zeros((LANES,),
                                                            jnp.float32)
            return 0
        lax.fori_loop(0, 2 * CHUNK, zrow, 0)

        def zblk(t, _):
            blk = s + ns * t

            @pl.when(blk < nzblk)
            def _():
                pltpu.sync_copy(mb0, agg.at[pl.ds(blk * 2 * CHUNK, 2 * CHUNK)])
            return 0
        lax.fori_loop(0, zps, zblk, 0)
        plsc.subcore_barrier()

        isets = ((sidx0, didx0, xdix0, isem0, xsem0),
                 (sidx1, didx1, xdix1, isem1, xsem1))
        gsets = ((kb0, qvb0, gsem0), (kb1, qvb1, gsem1))
        msets = ((mb0, msem0), (mb1, msem1))

        def i_copies(i, p):
            sidx, didx, _, isem, _ = isets[p]
            sl = pl.ds(ebase + i * CHUNK, CHUNK)
            return (
                pltpu.make_async_copy(src_hbm.at[sl], sidx, isem),
                pltpu.make_async_copy(dst_hbm.at[sl], didx, isem),
            )

        def x_copy(t, mp):
            xdix, xsem = isets[mp][2], isets[mp][4]
            sl = pl.ds(ebase + t * 2 * CHUNK, 2 * CHUNK)
            return pltpu.make_async_copy(dst_hbm.at[sl], xdix, xsem)

        def g_copies(p):
            sidx, didx = isets[p][0], isets[p][1]
            kb, qvb, gsem = gsets[p]
            lo, hi = pl.ds(0, 16), pl.ds(16, CHUNK - 16)
            return (
                pltpu.make_async_copy(k_hbm.at[didx.at[lo]], kb.at[lo], gsem),
                pltpu.make_async_copy(k_hbm.at[didx.at[hi]], kb.at[hi], gsem),
                pltpu.make_async_copy(qv_hbm.at[sidx.at[lo]], qvb.at[lo],
                                      gsem),
                pltpu.make_async_copy(qv_hbm.at[sidx.at[hi]], qvb.at[hi],
                                      gsem),
            )

        def m_copy(mp):
            mb, msem = msets[mp]
            xdix = isets[mp][2]
            return pltpu.make_async_copy(mb, agg.at[xdix], msem)

        def compute(p, mp, half):
            kb, qvb, _ = gsets[p]
            mb, _ = msets[mp]
            himask = jnp.full((LANES,), -65536, jnp.int32)

            def crow(r, _):
                for u in range(D // LANES):
                    sl = pl.ds(LANES * u, LANES)
                    w = qvb[r, sl]
                    qf = lax.bitcast_convert_type(w << 16, jnp.float32)
                    vf = lax.bitcast_convert_type(w & himask, jnp.float32)
                    t = kb[r, sl] + qf
                    mb[half * CHUNK + r, sl] = vf / (1.0 + jnp.exp(-t))
                return 0
            lax.fori_loop(0, CHUNK, crow, 0)

        def superstep(u, t, mp, last):
            for cp in g_copies(0):
                cp.wait()

            @pl.when(u > 0)
            def _():
                m_copy(mp).wait()
            x_copy(t, mp).start()

            @pl.when(2 * t + 2 < nchunk)
            def _():
                for cp in i_copies(2 * t + 2, 0):
                    cp.start()
            compute(0, mp, 0)

            @pl.when(2 * t + 2 < nchunk)
            def _():
                for cp in i_copies(2 * t + 2, 0):
                    cp.wait()
                for cp in g_copies(0):
                    cp.start()
            for cp in g_copies(1):
                cp.wait()

            @pl.when(2 * t + 3 < nchunk)
            def _():
                for cp in i_copies(2 * t + 3, 1):
                    cp.start()
            compute(1, mp, 1)

            @pl.when(2 * t + 3 < nchunk)
            def _():
                for cp in i_copies(2 * t + 3, 1):
                    cp.wait()
                for cp in g_copies(1):
                    cp.start()
            x_copy(t, mp).wait()
            m_copy(mp).start(add=True)

        for p in (0, 1):
            for cp in i_copies(p, p):
                cp.start()
            for cp in i_copies(p, p):
                cp.wait()
            for cp in g_copies(p):
                cp.start()

        def superpair(u, _):
            superstep(u, 2 * u, 0, False)
            superstep(u, 2 * u + 1, 1, False)
            return 0
        lax.fori_loop(0, nsuper // 2, superpair, 0)
        superstep(nsuper // 2, nsuper - 1, 0, True)
        m_copy(1).wait()
        m_copy(0).wait()
        plsc.subcore_barrier()

        for t in range(bps):
            blk = s + ns * t

            @pl.when(blk < nblk)
            def _():
                pltpu.sync_copy(agg.at[pl.ds(blk * ZC, ZC)],
                                out_hbm.at[c, pl.ds(blk * ZC, ZC)])

    return pl.kernel(
        body,
        out_type=jax.ShapeDtypeStruct((nc, N, D), jnp.float32),
        mesh=plsc.VectorSubcoreMesh(core_axis_name="c", subcore_axis_name="s"),
        scratch_types=[
            pltpu.VMEM((CHUNK,), jnp.int32),
            pltpu.VMEM((CHUNK,), jnp.int32),
            pltpu.VMEM((2 * CHUNK,), jnp.int32),
            pltpu.VMEM((CHUNK,), jnp.int32),
            pltpu.VMEM((CHUNK,), jnp.int32),
            pltpu.VMEM((2 * CHUNK,), jnp.int32),
            pltpu.VMEM((CHUNK, D), jnp.float32),
            pltpu.VMEM((CHUNK, D), jnp.int32),
            pltpu.VMEM((CHUNK, D), jnp.float32),
            pltpu.VMEM((CHUNK, D), jnp.int32),
            pltpu.VMEM((2 * CHUNK, D), jnp.float32),
            pltpu.VMEM((2 * CHUNK, D), jnp.float32),
            pltpu.VMEM_SHARED((N, D), jnp.float32),
            pltpu.SemaphoreType.DMA,
            pltpu.SemaphoreType.DMA,
            pltpu.SemaphoreType.DMA,
            pltpu.SemaphoreType.DMA,
            pltpu.SemaphoreType.DMA,
            pltpu.SemaphoreType.DMA,
            pltpu.SemaphoreType.DMA,
            pltpu.SemaphoreType.DMA,
        ],
    )


def _edges(src, dst, k, qv):
    mesh = plsc.VectorSubcoreMesh(core_axis_name="c", subcore_axis_name="s")
    f = _make_edge_kernel(mesh.num_cores, mesh.num_subcores)
    return f(src, dst, k, qv)


def kernel(x, edge_index, batch, Wk0, bk0, Wq0, bq0, Wv0, bv0, Ws0, b0, g0, be0,
           Wk1, bk1, Wq1, bq1, Wv1, bv1, Ws1, b1, g1, be1):
    src = edge_index[0]
    dst = edge_index[1]
    w0 = jnp.concatenate([Wk0.T, Wq0.T, Wv0.T, Ws0.T], axis=1)
    b0cat = jnp.concatenate([bk0, bq0, bv0, b0]).reshape(1, 4 * D)
    w1 = jnp.concatenate([Wk1.T, Wq1.T, Wv1.T, Ws1.T], axis=1)
    b1cat = jnp.concatenate([bk1, bq1, bv1, b1]).reshape(1, 4 * D)

    k0, qv0, s0 = _linear4(x, w0, b0cat)
    parts0 = _edges(src, dst, k0, qv0)
    y0, sum0, sq0 = _relu_stats(parts0, s0)
    k1, qv1, s1 = _bn_linear4(y0, sum0, sq0, g0.reshape(1, D),
                              be0.reshape(1, D), w1, b1cat)
    parts1 = _edges(src, dst, k1, qv1)
    y1, sum1, sq1 = _relu_stats(parts1, s1)
    return _bn_final(y1, sum1, sq1, g1.reshape(1, D), be1.reshape(1, D))


# speedup vs baseline: 1.0211x; 1.0211x over previous
import jax
import jax.numpy as jnp
from jax import lax
from jax.experimental import pallas as pl
from jax.experimental.pallas import tpu as pltpu
from jax.experimental.pallas import tpu_sc as plsc

N, E, D = 10000, 320000, 128
EPS = 1e-5
ROWS = 2000
CHUNK = 40
ZC = 200
LANES = 16


def _lin_body(x_ref, w_ref, b_ref, k_ref, qv_ref, s_ref):
    acc = jnp.dot(x_ref[...], w_ref[...],
                  preferred_element_type=jnp.float32) + b_ref[...]
    k_ref[...] = acc[:, 0:D]
    qlo = lax.bitcast_convert_type(acc[:, D:2 * D].astype(jnp.bfloat16),
                                   jnp.uint16).astype(jnp.int32)
    vhi = lax.bitcast_convert_type(acc[:, 2 * D:3 * D].astype(jnp.bfloat16),
                                   jnp.uint16).astype(jnp.int32)
    qv_ref[...] = (vhi << 16) | qlo
    s_ref[...] = acc[:, 3 * D:4 * D]


_LIN_OUT_SPECS = [
    pl.BlockSpec((ROWS, D), lambda i: (i, 0)),
    pl.BlockSpec((ROWS, D), lambda i: (i, 0)),
    pl.BlockSpec((ROWS, D), lambda i: (i, 0)),
]
_LIN_OUT_SHAPE = [
    jax.ShapeDtypeStruct((N, D), jnp.float32),
    jax.ShapeDtypeStruct((N, D), jnp.int32),
    jax.ShapeDtypeStruct((N, D), jnp.float32),
]


def _linear4(x, wcat, bcat):
    return pl.pallas_call(
        _lin_body,
        grid=(N // ROWS,),
        in_specs=[
            pl.BlockSpec((ROWS, D), lambda i: (i, 0)),
            pl.BlockSpec((D, 4 * D), lambda i: (0, 0)),
            pl.BlockSpec((1, 4 * D), lambda i: (0, 0)),
        ],
        out_specs=_LIN_OUT_SPECS,
        out_shape=_LIN_OUT_SHAPE,
    )(x, wcat, bcat)


PH = N // ROWS


def _accum_stats(i, y, ybuf, stat):
    ybuf[pl.ds(i * ROWS, ROWS), :] = y

    @pl.when(i == 0)
    def _():
        stat[...] = jnp.zeros_like(stat)

    stat[...] += jnp.concatenate(
        [jnp.sum(y, axis=0, keepdims=True),
         jnp.sum(y * y, axis=0, keepdims=True),
         jnp.zeros((6, D), jnp.float32)], axis=0)


def _bn_from_stats(j, ybuf, stat, g_ref, be_ref):
    mean = stat[0:1, :] * (1.0 / N)
    var = stat[1:2, :] * (1.0 / N) - mean * mean
    inv = lax.rsqrt(var + EPS)
    y = ybuf[pl.ds(j * ROWS, ROWS), :]
    return (y - mean) * (inv * g_ref[...]) + be_ref[...]


def _mid_body(a_ref, s_ref, g_ref, be_ref, w_ref, b_ref,
              k_ref, qv_ref, s2_ref, ybuf, stat):
    i = pl.program_id(0)

    @pl.when(i < PH)
    def _():
        y = jnp.maximum(a_ref[0] + a_ref[1] + s_ref[...], 0.0)
        _accum_stats(i, y, ybuf, stat)

    @pl.when(i >= PH)
    def _():
        h = _bn_from_stats(i - PH, ybuf, stat, g_ref, be_ref)
        acc = jnp.dot(h, w_ref[...],
                      preferred_element_type=jnp.float32) + b_ref[...]
        k_ref[...] = acc[:, 0:D]
        qlo = lax.bitcast_convert_type(acc[:, D:2 * D].astype(jnp.bfloat16),
                                       jnp.uint16).astype(jnp.int32)
        vhi = lax.bitcast_convert_type(
            acc[:, 2 * D:3 * D].astype(jnp.bfloat16),
            jnp.uint16).astype(jnp.int32)
        qv_ref[...] = (vhi << 16) | qlo
        s2_ref[...] = acc[:, 3 * D:4 * D]


def _p1_map(i):
    return (jnp.where(i < PH, i, PH - 1), 0)


def _p1_map3(i):
    return (0, jnp.where(i < PH, i, PH - 1), 0)


def _p2_map(i):
    return (jnp.where(i < PH, 0, i - PH), 0)


def _relu_bn_linear4(agg_parts, s, g, be, wcat, bcat):
    return pl.pallas_call(
        _mid_body,
        grid=(2 * PH,),
        in_specs=[
            pl.BlockSpec((2, ROWS, D), _p1_map3),
            pl.BlockSpec((ROWS, D), _p1_map),
            pl.BlockSpec((1, D), lambda i: (0, 0)),
            pl.BlockSpec((1, D), lambda i: (0, 0)),
            pl.BlockSpec((D, 4 * D), lambda i: (0, 0)),
            pl.BlockSpec((1, 4 * D), lambda i: (0, 0)),
        ],
        out_specs=[
            pl.BlockSpec((ROWS, D), _p2_map),
            pl.BlockSpec((ROWS, D), _p2_map),
            pl.BlockSpec((ROWS, D), _p2_map),
        ],
        out_shape=_LIN_OUT_SHAPE,
        scratch_shapes=[
            pltpu.VMEM((N, D), jnp.float32),
            pltpu.VMEM((8, D), jnp.float32),
        ],
    )(agg_parts, s, g, be, wcat, bcat)


def _fin_body(a_ref, s_ref, g_ref, be_ref, out_ref, ybuf, stat):
    i = pl.program_id(0)

    @pl.when(i < PH)
    def _():
        y = jnp.maximum(a_ref[0] + a_ref[1] + s_ref[...], 0.0)
        _accum_stats(i, y, ybuf, stat)

    @pl.when(i >= PH)
    def _():
        out_ref[...] = _bn_from_stats(i - PH, ybuf, stat, g_ref, be_ref)


def _relu_bn_final(agg_parts, s, g, be):
    return pl.pallas_call(
        _fin_body,
        grid=(2 * PH,),
        in_specs=[
            pl.BlockSpec((2, ROWS, D), _p1_map3),
            pl.BlockSpec((ROWS, D), _p1_map),
            pl.BlockSpec((1, D), lambda i: (0, 0)),
            pl.BlockSpec((1, D), lambda i: (0, 0)),
        ],
        out_specs=pl.BlockSpec((ROWS, D), _p2_map),
        out_shape=jax.ShapeDtypeStruct((N, D), jnp.float32),
        scratch_shapes=[
            pltpu.VMEM((N, D), jnp.float32),
            pltpu.VMEM((8, D), jnp.float32),
        ],
    )(agg_parts, s, g, be)


def _make_edge_kernel(nc, ns):
    npw = nc * ns
    epw = E // npw
    nchunk = epw // CHUNK
    nsuper = nchunk // 2
    nblk = N // ZC
    bps = (nblk + ns - 1) // ns
    nzblk = N // (2 * CHUNK)
    zps = (nzblk + ns - 1) // ns

    def body(src_hbm, dst_hbm, k_hbm, qv_hbm, out_hbm,
             sidx0, didx0, xdix0, sidx1, didx1, xdix1,
             kb0, qvb0, kb1, qvb1, mb0, mb1, agg,
             gsem0, gsem1, msem0, msem1, isem0, isem1, xsem0, xsem1):
        c = lax.axis_index("c")
        s = lax.axis_index("s")
        wid = s * nc + c
        ebase = wid * epw

        def zrow(r, _):
            for g in range(D // LANES):
                mb0[r, pl.ds(g * LANES, LANES)] = jnp.zeros((LANES,),
                                                            jnp.float32)
            return 0
        lax.fori_loop(0, 2 * CHUNK, zrow, 0)

        def zblk(t, _):
            blk = s + ns * t

            @pl.when(blk < nzblk)
            def _():
                pltpu.sync_copy(mb0, agg.at[pl.ds(blk * 2 * CHUNK, 2 * CHUNK)])
            return 0
        lax.fori_loop(0, zps, zblk, 0)
        plsc.subcore_barrier()

        isets = ((sidx0, didx0, xdix0, isem0, xsem0),
                 (sidx1, didx1, xdix1, isem1, xsem1))
        gsets = ((kb0, qvb0, gsem0), (kb1, qvb1, gsem1))
        msets = ((mb0, msem0), (mb1, msem1))

        def i_copies(i, p):
            sidx, didx, _, isem, _ = isets[p]
            sl = pl.ds(ebase + i * CHUNK, CHUNK)
            return (
                pltpu.make_async_copy(src_hbm.at[sl], sidx, isem),
                pltpu.make_async_copy(dst_hbm.at[sl], didx, isem),
            )

        def x_copy(t, mp):
            xdix, xsem = isets[mp][2], isets[mp][4]
            sl = pl.ds(ebase + t * 2 * CHUNK, 2 * CHUNK)
            return pltpu.make_async_copy(dst_hbm.at[sl], xdix, xsem)

        def g_copies(p):
            sidx, didx = isets[p][0], isets[p][1]
            kb, qvb, gsem = gsets[p]
            return (
                pltpu.make_async_copy(k_hbm.at[didx], kb, gsem),
                pltpu.make_async_copy(qv_hbm.at[sidx], qvb, gsem),
            )

        def m_copy(mp):
            mb, msem = msets[mp]
            xdix = isets[mp][2]
            return pltpu.make_async_copy(mb, agg.at[xdix], msem)

        def compute(p, mp, half):
            kb, qvb, _ = gsets[p]
            mb, _ = msets[mp]
            himask = jnp.full((LANES,), -65536, jnp.int32)

            def crow(r, _):
                for u in range(D // LANES):
                    sl = pl.ds(LANES * u, LANES)
                    w = qvb[r, sl]
                    qf = lax.bitcast_convert_type(w << 16, jnp.float32)
                    vf = lax.bitcast_convert_type(w & himask, jnp.float32)
                    t = kb[r, sl] + qf
                    mb[half * CHUNK + r, sl] = vf / (1.0 + jnp.exp(-t))
                return 0
            lax.fori_loop(0, CHUNK, crow, 0)

        def superstep(u, t, mp, last):
            for cp in g_copies(0):
                cp.wait()

            @pl.when(u > 0)
            def _():
                m_copy(mp).wait()
            x_copy(t, mp).start()

            @pl.when(2 * t + 2 < nchunk)
            def _():
                for cp in i_copies(2 * t + 2, 0):
                    cp.start()
            compute(0, mp, 0)

            @pl.when(2 * t + 2 < nchunk)
            def _():
                for cp in i_copies(2 * t + 2, 0):
                    cp.wait()
                for cp in g_copies(0):
                    cp.start()
            for cp in g_copies(1):
                cp.wait()

            @pl.when(2 * t + 3 < nchunk)
            def _():
                for cp in i_copies(2 * t + 3, 1):
                    cp.start()
            compute(1, mp, 1)

            @pl.when(2 * t + 3 < nchunk)
            def _():
                for cp in i_copies(2 * t + 3, 1):
                    cp.wait()
                for cp in g_copies(1):
                    cp.start()
            x_copy(t, mp).wait()
            m_copy(mp).start(add=True)

        for p in (0, 1):
            for cp in i_copies(p, p):
                cp.start()
            for cp in i_copies(p, p):
                cp.wait()
            for cp in g_copies(p):
                cp.start()

        def superpair(u, _):
            superstep(u, 2 * u, 0, False)
            superstep(u, 2 * u + 1, 1, False)
            return 0
        lax.fori_loop(0, nsuper // 2, superpair, 0)
        superstep(nsuper // 2, nsuper - 1, 0, True)
        m_copy(1).wait()
        m_copy(0).wait()
        plsc.subcore_barrier()

        for t in range(bps):
            blk = s + ns * t

            @pl.when(blk < nblk)
            def _():
                pltpu.sync_copy(agg.at[pl.ds(blk * ZC, ZC)],
                                out_hbm.at[c, pl.ds(blk * ZC, ZC)])

    return pl.kernel(
        body,
        out_type=jax.ShapeDtypeStruct((nc, N, D), jnp.float32),
        mesh=plsc.VectorSubcoreMesh(core_axis_name="c", subcore_axis_name="s"),
        scratch_types=[
            pltpu.VMEM((CHUNK,), jnp.int32),
            pltpu.VMEM((CHUNK,), jnp.int32),
            pltpu.VMEM((2 * CHUNK,), jnp.int32),
            pltpu.VMEM((CHUNK,), jnp.int32),
            pltpu.VMEM((CHUNK,), jnp.int32),
            pltpu.VMEM((2 * CHUNK,), jnp.int32),
            pltpu.VMEM((CHUNK, D), jnp.float32),
            pltpu.VMEM((CHUNK, D), jnp.int32),
            pltpu.VMEM((CHUNK, D), jnp.float32),
            pltpu.VMEM((CHUNK, D), jnp.int32),
            pltpu.VMEM((2 * CHUNK, D), jnp.float32),
            pltpu.VMEM((2 * CHUNK, D), jnp.float32),
            pltpu.VMEM_SHARED((N, D), jnp.float32),
            pltpu.SemaphoreType.DMA,
            pltpu.SemaphoreType.DMA,
            pltpu.SemaphoreType.DMA,
            pltpu.SemaphoreType.DMA,
            pltpu.SemaphoreType.DMA,
            pltpu.SemaphoreType.DMA,
            pltpu.SemaphoreType.DMA,
            pltpu.SemaphoreType.DMA,
        ],
    )


def _edges(src, dst, k, qv):
    mesh = plsc.VectorSubcoreMesh(core_axis_name="c", subcore_axis_name="s")
    f = _make_edge_kernel(mesh.num_cores, mesh.num_subcores)
    return f(src, dst, k, qv)


def kernel(x, edge_index, batch, Wk0, bk0, Wq0, bq0, Wv0, bv0, Ws0, b0, g0, be0,
           Wk1, bk1, Wq1, bq1, Wv1, bv1, Ws1, b1, g1, be1):
    src = edge_index[0]
    dst = edge_index[1]
    w0 = jnp.concatenate([Wk0.T, Wq0.T, Wv0.T, Ws0.T], axis=1)
    b0cat = jnp.concatenate([bk0, bq0, bv0, b0]).reshape(1, 4 * D)
    w1 = jnp.concatenate([Wk1.T, Wq1.T, Wv1.T, Ws1.T], axis=1)
    b1cat = jnp.concatenate([bk1, bq1, bv1, b1]).reshape(1, 4 * D)

    k0, qv0, s0 = _linear4(x, w0, b0cat)
    parts0 = _edges(src, dst, k0, qv0)
    k1, qv1, s1 = _relu_bn_linear4(parts0, s0, g0.reshape(1, D),
                                   be0.reshape(1, D), w1, b1cat)
    parts1 = _edges(src, dst, k1, qv1)
    return _relu_bn_final(parts1, s1, g1.reshape(1, D), be1.reshape(1, D))
